# baseline (speedup 1.0000x reference)
import functools

import jax
import jax.numpy as jnp
from jax import lax
from jax.experimental import pallas as pl
from jax.experimental.pallas import tpu as pltpu
from jax.experimental.pallas import tpu_sc as plsc

N_NODES = 10000
FEAT = 128
HEADS = 8
CH = 8
E_EDGES = 320000

NC = 2
NS = 16

N_PAD = 10112
E_PAD = 331776
CHUNK = 64
ROWS_PT = N_PAD // NS
D = 128

_HI = lax.Precision.HIGHEST

_GDN = lax.GatherDimensionNumbers(
    offset_dims=(), collapsed_slice_dims=(0,), start_index_map=(0,))


def _vperm(v, idx):
    return lax.gather(v, idx[:, None], dimension_numbers=_GDN, slice_sizes=(1,),
                      mode=lax.GatherScatterMode.PROMISE_IN_BOUNDS)


def _dense1_body(x_ref, w1_ref, g1_ref, md_ref, table_ref, ad_ref):
    h = jnp.dot(x_ref[...], w1_ref[...], precision=_HI,
                preferred_element_type=jnp.float32)
    table_ref[...] = jnp.dot(h, g1_ref[...], precision=_HI,
                             preferred_element_type=jnp.float32)
    ad_ref[...] = jnp.dot(h, md_ref[...], precision=_HI,
                          preferred_element_type=jnp.float32)


def _dense2_body(acca_ref, accb_ref, rsel_ref, b1_ref, w2_ref, g2a_ref, g2b_ref,
                 a2d_ref, ta_ref, tb_ref, ad_ref):
    a = acca_ref[...] + accb_ref[...]
    srep = jnp.dot(a, rsel_ref[...], precision=_HI,
                   preferred_element_type=jnp.float32)
    m = a[:, :HEADS * CH] / (srep + 1e-16) + b1_ref[...]
    m = jnp.where(m > 0, m, jnp.exp(m) - 1.0)
    h2 = jnp.dot(m, w2_ref[...], precision=_HI,
                 preferred_element_type=jnp.float32)
    ta_ref[...] = jnp.dot(h2, g2a_ref[...], precision=_HI,
                          preferred_element_type=jnp.float32)
    tb_ref[...] = jnp.dot(h2, g2b_ref[...], precision=_HI,
                          preferred_element_type=jnp.float32)
    ad_ref[...] = jnp.dot(h2, a2d_ref[...], precision=_HI,
                          preferred_element_type=jnp.float32)


def _final_body(acca_ref, accb_ref, p0_ref, p1_ref, b2_ref, out_ref):
    a0 = acca_ref[...]
    a1 = accb_ref[...]
    o0 = a0[:, :64] / (a0[:, 64:65] + 1e-16)
    o1 = a1[:, :64] / (a1[:, 64:65] + 1e-16)
    out_ref[...] = (jnp.dot(o0, p0_ref[...], precision=_HI,
                            preferred_element_type=jnp.float32)
                    + jnp.dot(o1, p1_ref[...], precision=_HI,
                              preferred_element_type=jnp.float32)
                    + b2_ref[...])


def _edge_pass(H, C, table, src_p, dst_p, a_d, zeros, split_edges):
    mesh = plsc.VectorSubcoreMesh(core_axis_name="c", subcore_axis_name="s")
    nwork = NC * NS if split_edges else NS
    ept = E_PAD // nwork
    nchunks = ept // CHUNK
    assert nchunks % 3 == 0

    @functools.partial(
        pl.kernel, mesh=mesh,
        out_type=jax.ShapeDtypeStruct((NC, N_PAD, D), jnp.float32),
        scratch_types=[
            pltpu.VMEM((CHUNK,), jnp.int32), pltpu.VMEM((CHUNK,), jnp.int32),
            pltpu.VMEM((CHUNK,), jnp.int32),
            pltpu.VMEM((CHUNK,), jnp.int32), pltpu.VMEM((CHUNK,), jnp.int32),
            pltpu.VMEM((CHUNK,), jnp.int32),
            pltpu.VMEM((CHUNK, D), jnp.float32), pltpu.VMEM((CHUNK, D), jnp.float32),
            pltpu.VMEM((CHUNK, D), jnp.float32),
            pltpu.VMEM((CHUNK, D), jnp.float32), pltpu.VMEM((CHUNK, D), jnp.float32),
            pltpu.VMEM((CHUNK, D), jnp.float32),
            pltpu.VMEM_SHARED((N_PAD, D), jnp.float32),
            pltpu.SemaphoreType.DMA, pltpu.SemaphoreType.DMA,
            pltpu.SemaphoreType.DMA,
        ],
    )
    def k(table_h, src_h, dst_h, ad_h, z_h, out_h,
          i0, i1, i2, ds0, ds1, ds2, r0, r1, r2, a0, a1, a2, acc, g0, g1, g2):
        srcb, dstb = [i0, i1, i2], [ds0, ds1, ds2]
        rowsb, adrb, gsem = [r0, r1, r2], [a0, a1, a2], [g0, g1, g2]
        cid = lax.axis_index("c")
        sid = lax.axis_index("s")
        wid = sid * NC + cid if split_edges else sid
        off = cid * N_NODES

        def stage(c, b):
            base = pl.multiple_of(wid * ept + c * CHUNK, CHUNK)
            pltpu.sync_copy(src_h.at[pl.ds(base, CHUNK)], srcb[b])
            pltpu.sync_copy(dst_h.at[pl.ds(base, CHUNK)], dstb[b])
            if not split_edges:
                for g in range(CHUNK // 16):
                    srcb[b][pl.ds(g * 16, 16)] = srcb[b][pl.ds(g * 16, 16)] + off
            pltpu.async_copy(table_h.at[srcb[b]], rowsb[b], gsem[b])
            pltpu.async_copy(ad_h.at[dstb[b]], adrb[b], gsem[b])

        def wait_gathers(b):
            pltpu.make_async_copy(table_h.at[srcb[b]], rowsb[b], gsem[b]).wait()
            pltpu.make_async_copy(ad_h.at[dstb[b]], adrb[b], gsem[b]).wait()

        row0 = pl.multiple_of(sid * ROWS_PT, 8)
        pltpu.sync_copy(z_h.at[pl.ds(row0, ROWS_PT)],
                        acc.at[pl.ds(row0, ROWS_PT)])
        plsc.subcore_barrier()
        log2c = C.bit_length() - 1
        stage(0, 0)
        stage(1, 1)

        def body(cc, carry):
            iota = lax.iota(jnp.int32, 16)
            pmask = jnp.where(iota < H, 1.0, 0.0)
            pidx = [(iota + 16 * j) >> log2c for j in range(H * C // 16)]
            for b in (0, 1, 2):
                c = 3 * cc + b
                fb = (b + 2) % 3
                stage(c + 2, fb)
                wait_gathers(b)
                rows_v, adr_v = rowsb[b], adrb[b]
                for i in range(CHUNK):
                    ad = adr_v[i, pl.ds(0, 16)]
                    as_ = rows_v[i, pl.ds(H * C, 16)]
                    e = as_ + ad
                    e = jnp.where(e > 0, e, 0.2 * e)
                    p = jnp.exp(e) * pmask
                    rows_v[i, pl.ds(H * C, 16)] = p
                    for j in range(H * C // 16):
                        pv = _vperm(p, pidx[j])
                        v = rows_v[i, pl.ds(16 * j, 16)]
                        rows_v[i, pl.ds(16 * j, 16)] = v * pv
                pltpu.sync_copy(rows_v, acc.at[dstb[b]], add=True)
            return carry

        lax.fori_loop(0, nchunks // 3, body, 0)
        wait_gathers(0)
        wait_gathers(1)
        plsc.subcore_barrier()
        pltpu.sync_copy(acc.at[pl.ds(row0, ROWS_PT)],
                        out_h.at[cid, pl.ds(row0, ROWS_PT)])

    return k(table, src_p, dst_p, a_d, zeros)


def kernel(x, edge_index, W1, att_src1, att_dst1, b1, W2, att_src2, att_dst2, b2):
    f32 = jnp.float32
    src = edge_index[0]
    dst = edge_index[1]
    pad_e = E_PAD + 2 * CHUNK - E_EDGES
    src_p = jnp.concatenate([src, jnp.zeros((pad_e,), jnp.int32)])
    dst_p = jnp.concatenate(
        [dst, N_NODES + (jnp.arange(pad_e, dtype=jnp.int32) % 100)])

    eyeH = jnp.eye(HEADS, dtype=f32)
    eye64 = jnp.eye(64, dtype=f32)
    m1s = (att_src1[:, :, None] * eyeH[:, None, :]).reshape(HEADS * CH, HEADS)
    m1d = (att_dst1[:, :, None] * eyeH[:, None, :]).reshape(HEADS * CH, HEADS)
    g1 = jnp.concatenate([eye64, m1s, jnp.zeros((64, 56), f32)], axis=1)
    rsel = jnp.concatenate(
        [jnp.zeros((64, 64), f32),
         jnp.repeat(eyeH, CH, axis=1),
         jnp.zeros((56, 64), f32)], axis=0)
    top = jnp.concatenate([eye64, jnp.zeros((64, 64), f32)], axis=0)
    bot = jnp.concatenate([jnp.zeros((64, 64), f32), eye64], axis=0)
    g2a = jnp.concatenate([top, att_src2.T, jnp.zeros((FEAT, 63), f32)], axis=1)
    g2b = jnp.concatenate([bot, att_src2.T, jnp.zeros((FEAT, 63), f32)], axis=1)
    a2d = att_dst2.T
    p0 = top.T
    p1 = bot.T

    table1, ad1 = pl.pallas_call(
        _dense1_body,
        out_shape=(jax.ShapeDtypeStruct((N_NODES, D), f32),
                   jax.ShapeDtypeStruct((N_NODES, HEADS), f32)),
    )(x, W1, g1, m1d)
    ad1p = jnp.zeros((N_PAD, D), f32).at[:N_NODES, :HEADS].set(ad1)
    zpad = jnp.zeros((N_PAD, D), f32)

    acc1 = _edge_pass(HEADS, CH, table1, src_p, dst_p, ad1p, zpad,
                      split_edges=True)

    t2a, t2b, ad2 = pl.pallas_call(
        _dense2_body,
        out_shape=(jax.ShapeDtypeStruct((N_NODES, D), f32),
                   jax.ShapeDtypeStruct((N_NODES, D), f32),
                   jax.ShapeDtypeStruct((N_NODES, 1), f32)),
    )(acc1[0, :N_NODES], acc1[1, :N_NODES], rsel, b1.reshape(1, -1), W2,
      g2a, g2b, a2d)
    table2 = jnp.concatenate([t2a, t2b], axis=0)
    ad2p = jnp.zeros((N_PAD, D), f32).at[:N_NODES, :1].set(ad2)

    acc2 = _edge_pass(1, 64, table2, src_p, dst_p, ad2p, zpad,
                      split_edges=False)

    out = pl.pallas_call(
        _final_body,
        out_shape=jax.ShapeDtypeStruct((N_NODES, FEAT), f32),
    )(acc2[0, :N_NODES], acc2[1, :N_NODES], p0, p1, b2.reshape(1, -1))
    return out


# speedup vs baseline: 1.0143x; 1.0143x over previous
import functools

import jax
import jax.numpy as jnp
from jax import lax
from jax.experimental import pallas as pl
from jax.experimental.pallas import tpu as pltpu
from jax.experimental.pallas import tpu_sc as plsc

N_NODES = 10000
FEAT = 128
HEADS = 8
CH = 8
E_EDGES = 320000

NC = 2
NS = 16

N_PAD = 10112
E_PAD = 331776
CHUNK = 96
ROWS_PT = N_PAD // NS
D = 128

_HI = lax.Precision.HIGHEST

_GDN = lax.GatherDimensionNumbers(
    offset_dims=(), collapsed_slice_dims=(0,), start_index_map=(0,))


def _vperm(v, idx):
    return lax.gather(v, idx[:, None], dimension_numbers=_GDN, slice_sizes=(1,),
                      mode=lax.GatherScatterMode.PROMISE_IN_BOUNDS)


def _dense1_body(x_ref, w1_ref, g1_ref, md_ref, table_ref, ad_ref):
    h = jnp.dot(x_ref[...], w1_ref[...], precision=_HI,
                preferred_element_type=jnp.float32)
    table_ref[...] = jnp.dot(h, g1_ref[...], precision=_HI,
                             preferred_element_type=jnp.float32)
    ad_ref[...] = jnp.dot(h, md_ref[...], precision=_HI,
                          preferred_element_type=jnp.float32)


def _dense2_body(acca_ref, accb_ref, rsel_ref, b1_ref, w2_ref, g2a_ref, g2b_ref,
                 a2d_ref, ta_ref, tb_ref, ad_ref):
    a = acca_ref[...] + accb_ref[...]
    srep = jnp.dot(a, rsel_ref[...], precision=_HI,
                   preferred_element_type=jnp.float32)
    m = a[:, :HEADS * CH] / (srep + 1e-16) + b1_ref[...]
    m = jnp.where(m > 0, m, jnp.exp(m) - 1.0)
    h2 = jnp.dot(m, w2_ref[...], precision=_HI,
                 preferred_element_type=jnp.float32)
    ta_ref[...] = jnp.dot(h2, g2a_ref[...], precision=_HI,
                          preferred_element_type=jnp.float32)
    tb_ref[...] = jnp.dot(h2, g2b_ref[...], precision=_HI,
                          preferred_element_type=jnp.float32)
    ad_ref[...] = jnp.dot(h2, a2d_ref[...], precision=_HI,
                          preferred_element_type=jnp.float32)


def _final_body(acca_ref, accb_ref, p0_ref, p1_ref, b2_ref, out_ref):
    a0 = acca_ref[...]
    a1 = accb_ref[...]
    o0 = a0[:, :64] / (a0[:, 64:65] + 1e-16)
    o1 = a1[:, :64] / (a1[:, 64:65] + 1e-16)
    out_ref[...] = (jnp.dot(o0, p0_ref[...], precision=_HI,
                            preferred_element_type=jnp.float32)
                    + jnp.dot(o1, p1_ref[...], precision=_HI,
                              preferred_element_type=jnp.float32)
                    + b2_ref[...])


def _edge_pass(H, C, table, src_p, dst_p, a_d, zeros, split_edges):
    mesh = plsc.VectorSubcoreMesh(core_axis_name="c", subcore_axis_name="s")
    nwork = NC * NS if split_edges else NS
    ept = E_PAD // nwork
    nchunks = ept // CHUNK
    assert nchunks % 2 == 0

    @functools.partial(
        pl.kernel, mesh=mesh,
        out_type=jax.ShapeDtypeStruct((NC, N_PAD, D), jnp.float32),
        scratch_types=[
            pltpu.VMEM((CHUNK,), jnp.int32), pltpu.VMEM((CHUNK,), jnp.int32),
            pltpu.VMEM((CHUNK,), jnp.int32), pltpu.VMEM((CHUNK,), jnp.int32),
            pltpu.VMEM((CHUNK, D), jnp.float32), pltpu.VMEM((CHUNK, D), jnp.float32),
            pltpu.VMEM((CHUNK, D), jnp.float32), pltpu.VMEM((CHUNK, D), jnp.float32),
            pltpu.VMEM_SHARED((N_PAD, D), jnp.float32),
            pltpu.SemaphoreType.DMA, pltpu.SemaphoreType.DMA,
            pltpu.SemaphoreType.DMA, pltpu.SemaphoreType.DMA,
        ],
    )
    def k(table_h, src_h, dst_h, ad_h, z_h, out_h,
          s0, s1, d0, d1, r0, r1, a0, a1, acc, g0, g1, e0, e1):
        srcb, dstb, rowsb, adrb = [s0, s1], [d0, d1], [r0, r1], [a0, a1]
        gsem, ssem = [g0, g1], [e0, e1]
        cid = lax.axis_index("c")
        sid = lax.axis_index("s")
        wid = sid * NC + cid if split_edges else sid
        off = cid * N_NODES

        def stage(c, b):
            base = pl.multiple_of(wid * ept + c * CHUNK, CHUNK)
            pltpu.sync_copy(src_h.at[pl.ds(base, CHUNK)], srcb[b])
            pltpu.sync_copy(dst_h.at[pl.ds(base, CHUNK)], dstb[b])
            if not split_edges:
                for g in range(CHUNK // 16):
                    srcb[b][pl.ds(g * 16, 16)] = srcb[b][pl.ds(g * 16, 16)] + off
            pltpu.async_copy(table_h.at[srcb[b]], rowsb[b], gsem[b])
            pltpu.async_copy(ad_h.at[dstb[b]], adrb[b], gsem[b])

        def wait_gathers(b):
            pltpu.make_async_copy(table_h.at[srcb[b]], rowsb[b], gsem[b]).wait()
            pltpu.make_async_copy(ad_h.at[dstb[b]], adrb[b], gsem[b]).wait()

        def wait_scatter(b):
            pltpu.make_async_copy(rowsb[b], acc.at[dstb[b]], ssem[b]).wait()

        row0 = pl.multiple_of(sid * ROWS_PT, 8)
        pltpu.sync_copy(z_h.at[pl.ds(row0, ROWS_PT)],
                        acc.at[pl.ds(row0, ROWS_PT)])
        plsc.subcore_barrier()
        log2c = C.bit_length() - 1
        stage(0, 0)

        def body(cc, carry):
            iota = lax.iota(jnp.int32, 16)
            pmask = jnp.where(iota < H, 1.0, 0.0)
            pidx = [(iota + 16 * j) >> log2c for j in range(H * C // 16)]
            for b in (0, 1):
                c = 2 * cc + b
                nb = 1 - b
                if b == 0:
                    @pl.when(cc > 0)
                    def _():
                        wait_scatter(nb)
                else:
                    wait_scatter(nb)
                stage(c + 1, nb)
                wait_gathers(b)
                rows_v, adr_v = rowsb[b], adrb[b]
                for i in range(CHUNK):
                    ad = adr_v[i, pl.ds(0, 16)]
                    as_ = rows_v[i, pl.ds(H * C, 16)]
                    e = as_ + ad
                    e = jnp.where(e > 0, e, 0.2 * e)
                    p = jnp.exp(e) * pmask
                    rows_v[i, pl.ds(H * C, 16)] = p
                    for j in range(H * C // 16):
                        pv = _vperm(p, pidx[j])
                        v = rows_v[i, pl.ds(16 * j, 16)]
                        rows_v[i, pl.ds(16 * j, 16)] = v * pv
                pltpu.async_copy(rows_v, acc.at[dstb[b]], ssem[b], add=True)
            return carry

        lax.fori_loop(0, nchunks // 2, body, 0)
        wait_scatter(1)
        wait_gathers(0)
        plsc.subcore_barrier()
        pltpu.sync_copy(acc.at[pl.ds(row0, ROWS_PT)],
                        out_h.at[cid, pl.ds(row0, ROWS_PT)])

    return k(table, src_p, dst_p, a_d, zeros)


def kernel(x, edge_index, W1, att_src1, att_dst1, b1, W2, att_src2, att_dst2, b2):
    f32 = jnp.float32
    src = edge_index[0]
    dst = edge_index[1]
    pad_e = E_PAD + CHUNK - E_EDGES
    src_p = jnp.concatenate([src, jnp.zeros((pad_e,), jnp.int32)])
    dst_p = jnp.concatenate(
        [dst, N_NODES + (jnp.arange(pad_e, dtype=jnp.int32) % 100)])

    eyeH = jnp.eye(HEADS, dtype=f32)
    eye64 = jnp.eye(64, dtype=f32)
    m1s = (att_src1[:, :, None] * eyeH[:, None, :]).reshape(HEADS * CH, HEADS)
    m1d = (att_dst1[:, :, None] * eyeH[:, None, :]).reshape(HEADS * CH, HEADS)
    g1 = jnp.concatenate([eye64, m1s, jnp.zeros((64, 56), f32)], axis=1)
    rsel = jnp.concatenate(
        [jnp.zeros((64, 64), f32),
         jnp.repeat(eyeH, CH, axis=1),
         jnp.zeros((56, 64), f32)], axis=0)
    top = jnp.concatenate([eye64, jnp.zeros((64, 64), f32)], axis=0)
    bot = jnp.concatenate([jnp.zeros((64, 64), f32), eye64], axis=0)
    g2a = jnp.concatenate([top, att_src2.T, jnp.zeros((FEAT, 63), f32)], axis=1)
    g2b = jnp.concatenate([bot, att_src2.T, jnp.zeros((FEAT, 63), f32)], axis=1)
    a2d = att_dst2.T
    p0 = top.T
    p1 = bot.T

    table1, ad1 = pl.pallas_call(
        _dense1_body,
        out_shape=(jax.ShapeDtypeStruct((N_NODES, D), f32),
                   jax.ShapeDtypeStruct((N_NODES, HEADS), f32)),
    )(x, W1, g1, m1d)
    ad1p = jnp.zeros((N_PAD, D), f32).at[:N_NODES, :HEADS].set(ad1)
    zpad = jnp.zeros((N_PAD, D), f32)

    acc1 = _edge_pass(HEADS, CH, table1, src_p, dst_p, ad1p, zpad,
                      split_edges=True)

    t2a, t2b, ad2 = pl.pallas_call(
        _dense2_body,
        out_shape=(jax.ShapeDtypeStruct((N_NODES, D), f32),
                   jax.ShapeDtypeStruct((N_NODES, D), f32),
                   jax.ShapeDtypeStruct((N_NODES, 1), f32)),
    )(acc1[0, :N_NODES], acc1[1, :N_NODES], rsel, b1.reshape(1, -1), W2,
      g2a, g2b, a2d)
    table2 = jnp.concatenate([t2a, t2b], axis=0)
    ad2p = jnp.zeros((N_PAD, D), f32).at[:N_NODES, :1].set(ad2)

    acc2 = _edge_pass(1, 64, table2, src_p, dst_p, ad2p, zpad,
                      split_edges=False)

    out = pl.pallas_call(
        _final_body,
        out_shape=jax.ShapeDtypeStruct((N_NODES, FEAT), f32),
    )(acc2[0, :N_NODES], acc2[1, :N_NODES], p0, p1, b2.reshape(1, -1))
    return out


# speedup vs baseline: 1.1043x; 1.0888x over previous
import functools

import jax
import jax.numpy as jnp
from jax import lax
from jax.experimental import pallas as pl
from jax.experimental.pallas import tpu as pltpu
from jax.experimental.pallas import tpu_sc as plsc

N_NODES = 10000
FEAT = 128
HEADS = 8
CH = 8
E_EDGES = 320000

NC = 2
NS = 16

N_PAD = 10112
E_PAD = 327680
CHUNK = 32
ROWS_PT = N_PAD // NS
D = 128

_HI = lax.Precision.HIGHEST

_GDN = lax.GatherDimensionNumbers(
    offset_dims=(), collapsed_slice_dims=(0,), start_index_map=(0,))


def _vperm(v, idx):
    return lax.gather(v, idx[:, None], dimension_numbers=_GDN, slice_sizes=(1,),
                      mode=lax.GatherScatterMode.PROMISE_IN_BOUNDS)


def _dense1_body(x_ref, w1_ref, g1_ref, md_ref, table_ref, ad_ref):
    h = jnp.dot(x_ref[...], w1_ref[...], precision=_HI,
                preferred_element_type=jnp.float32)
    table_ref[...] = jnp.dot(h, g1_ref[...], precision=_HI,
                             preferred_element_type=jnp.float32)
    ad_ref[...] = jnp.dot(h, md_ref[...], precision=_HI,
                          preferred_element_type=jnp.float32)


def _dense2_body(acca_ref, accb_ref, rsel_ref, b1_ref, w2_ref, g2a_ref, g2b_ref,
                 a2d_ref, ta_ref, tb_ref, ad_ref):
    a = acca_ref[...] + accb_ref[...]
    srep = jnp.dot(a, rsel_ref[...], precision=_HI,
                   preferred_element_type=jnp.float32)
    m = a[:, :HEADS * CH] / (srep + 1e-16) + b1_ref[...]
    m = jnp.where(m > 0, m, jnp.exp(m) - 1.0)
    h2 = jnp.dot(m, w2_ref[...], precision=_HI,
                 preferred_element_type=jnp.float32)
    ta_ref[...] = jnp.dot(h2, g2a_ref[...], precision=_HI,
                          preferred_element_type=jnp.float32)
    tb_ref[...] = jnp.dot(h2, g2b_ref[...], precision=_HI,
                          preferred_element_type=jnp.float32)
    ad_ref[...] = jnp.dot(h2, a2d_ref[...], precision=_HI,
                          preferred_element_type=jnp.float32)


def _final_body(acca_ref, accb_ref, p0_ref, p1_ref, b2_ref, out_ref):
    a0 = acca_ref[...]
    a1 = accb_ref[...]
    o0 = a0[:, :64] / (a0[:, 64:65] + 1e-16)
    o1 = a1[:, :64] / (a1[:, 64:65] + 1e-16)
    out_ref[...] = (jnp.dot(o0, p0_ref[...], precision=_HI,
                            preferred_element_type=jnp.float32)
                    + jnp.dot(o1, p1_ref[...], precision=_HI,
                              preferred_element_type=jnp.float32)
                    + b2_ref[...])


def _edge_pass(H, C, table, src_p, dst_p, a_d, zeros, split_edges):
    mesh = plsc.VectorSubcoreMesh(core_axis_name="c", subcore_axis_name="s")
    nwork = NC * NS if split_edges else NS
    ept = E_PAD // nwork
    nchunks = ept // CHUNK
    assert nchunks % 2 == 0

    @functools.partial(
        pl.kernel, mesh=mesh,
        out_type=jax.ShapeDtypeStruct((NC, N_PAD, D), jnp.float32),
        scratch_types=[
            pltpu.VMEM((CHUNK,), jnp.int32), pltpu.VMEM((CHUNK,), jnp.int32),
            pltpu.VMEM((CHUNK,), jnp.int32), pltpu.VMEM((CHUNK,), jnp.int32),
            pltpu.VMEM((CHUNK, D), jnp.float32), pltpu.VMEM((CHUNK, D), jnp.float32),
            pltpu.VMEM((CHUNK, D), jnp.float32), pltpu.VMEM((CHUNK, D), jnp.float32),
            pltpu.VMEM_SHARED((N_PAD, D), jnp.float32),
            pltpu.SemaphoreType.DMA, pltpu.SemaphoreType.DMA,
            pltpu.SemaphoreType.DMA, pltpu.SemaphoreType.DMA,
        ],
    )
    def k(table_h, src_h, dst_h, ad_h, z_h, out_h,
          s0, s1, d0, d1, r0, r1, a0, a1, acc, g0, g1, e0, e1):
        srcb, dstb, rowsb, adrb = [s0, s1], [d0, d1], [r0, r1], [a0, a1]
        gsem, ssem = [g0, g1], [e0, e1]
        cid = lax.axis_index("c")
        sid = lax.axis_index("s")
        wid = sid * NC + cid if split_edges else sid
        off = cid * N_NODES

        def stage(c, b):
            base = pl.multiple_of(wid * ept + c * CHUNK, CHUNK)
            pltpu.sync_copy(src_h.at[pl.ds(base, CHUNK)], srcb[b])
            pltpu.sync_copy(dst_h.at[pl.ds(base, CHUNK)], dstb[b])
            if not split_edges:
                for g in range(CHUNK // 16):
                    srcb[b][pl.ds(g * 16, 16)] = srcb[b][pl.ds(g * 16, 16)] + off
            pltpu.async_copy(table_h.at[srcb[b]], rowsb[b], gsem[b])
            pltpu.async_copy(ad_h.at[dstb[b]], adrb[b], gsem[b])

        def wait_gathers(b):
            pltpu.make_async_copy(table_h.at[srcb[b]], rowsb[b], gsem[b]).wait()
            pltpu.make_async_copy(ad_h.at[dstb[b]], adrb[b], gsem[b]).wait()

        def wait_scatter(b):
            pltpu.make_async_copy(rowsb[b], acc.at[dstb[b]], ssem[b]).wait()

        row0 = pl.multiple_of(sid * ROWS_PT, 8)
        pltpu.sync_copy(z_h.at[pl.ds(row0, ROWS_PT)],
                        acc.at[pl.ds(row0, ROWS_PT)])
        plsc.subcore_barrier()
        log2c = C.bit_length() - 1
        stage(0, 0)

        def body(cc, carry):
            iota = lax.iota(jnp.int32, 16)
            pmask = jnp.where(iota < H, 1.0, 0.0)
            pidx = [(iota + 16 * j) >> log2c for j in range(H * C // 16)]
            for b in (0, 1):
                c = 2 * cc + b
                nb = 1 - b
                if b == 0:
                    @pl.when(cc > 0)
                    def _():
                        wait_scatter(nb)
                else:
                    wait_scatter(nb)
                stage(c + 1, nb)
                wait_gathers(b)
                rows_v, adr_v = rowsb[b], adrb[b]
                for i in range(CHUNK):
                    ad = adr_v[i, pl.ds(0, 16)]
                    as_ = rows_v[i, pl.ds(H * C, 16)]
                    e = as_ + ad
                    e = jnp.where(e > 0, e, 0.2 * e)
                    p = jnp.exp(e) * pmask
                    rows_v[i, pl.ds(H * C, 16)] = p
                    for j in range(H * C // 16):
                        pv = _vperm(p, pidx[j])
                        v = rows_v[i, pl.ds(16 * j, 16)]
                        rows_v[i, pl.ds(16 * j, 16)] = v * pv
                pltpu.async_copy(rows_v, acc.at[dstb[b]], ssem[b], add=True)
            return carry

        lax.fori_loop(0, nchunks // 2, body, 0)
        wait_scatter(1)
        wait_gathers(0)
        plsc.subcore_barrier()
        pltpu.sync_copy(acc.at[pl.ds(row0, ROWS_PT)],
                        out_h.at[cid, pl.ds(row0, ROWS_PT)])

    return k(table, src_p, dst_p, a_d, zeros)


def kernel(x, edge_index, W1, att_src1, att_dst1, b1, W2, att_src2, att_dst2, b2):
    f32 = jnp.float32
    src = edge_index[0]
    dst = edge_index[1]
    pad_e = E_PAD + CHUNK - E_EDGES
    src_p = jnp.concatenate([src, jnp.zeros((pad_e,), jnp.int32)])
    dst_p = jnp.concatenate(
        [dst, N_NODES + (jnp.arange(pad_e, dtype=jnp.int32) % 100)])

    eyeH = jnp.eye(HEADS, dtype=f32)
    eye64 = jnp.eye(64, dtype=f32)
    m1s = (att_src1[:, :, None] * eyeH[:, None, :]).reshape(HEADS * CH, HEADS)
    m1d = (att_dst1[:, :, None] * eyeH[:, None, :]).reshape(HEADS * CH, HEADS)
    g1 = jnp.concatenate([eye64, m1s, jnp.zeros((64, 56), f32)], axis=1)
    rsel = jnp.concatenate(
        [jnp.zeros((64, 64), f32),
         jnp.repeat(eyeH, CH, axis=1),
         jnp.zeros((56, 64), f32)], axis=0)
    top = jnp.concatenate([eye64, jnp.zeros((64, 64), f32)], axis=0)
    bot = jnp.concatenate([jnp.zeros((64, 64), f32), eye64], axis=0)
    g2a = jnp.concatenate([top, att_src2.T, jnp.zeros((FEAT, 63), f32)], axis=1)
    g2b = jnp.concatenate([bot, att_src2.T, jnp.zeros((FEAT, 63), f32)], axis=1)
    a2d = att_dst2.T
    p0 = top.T
    p1 = bot.T

    table1, ad1 = pl.pallas_call(
        _dense1_body,
        out_shape=(jax.ShapeDtypeStruct((N_NODES, D), f32),
                   jax.ShapeDtypeStruct((N_NODES, HEADS), f32)),
    )(x, W1, g1, m1d)
    ad1p = jnp.zeros((N_PAD, D), f32).at[:N_NODES, :HEADS].set(ad1)
    zpad = jnp.zeros((N_PAD, D), f32)

    acc1 = _edge_pass(HEADS, CH, table1, src_p, dst_p, ad1p, zpad,
                      split_edges=True)

    t2a, t2b, ad2 = pl.pallas_call(
        _dense2_body,
        out_shape=(jax.ShapeDtypeStruct((N_NODES, D), f32),
                   jax.ShapeDtypeStruct((N_NODES, D), f32),
                   jax.ShapeDtypeStruct((N_NODES, 1), f32)),
    )(acc1[0, :N_NODES], acc1[1, :N_NODES], rsel, b1.reshape(1, -1), W2,
      g2a, g2b, a2d)
    table2 = jnp.concatenate([t2a, t2b], axis=0)
    ad2p = jnp.zeros((N_PAD, D), f32).at[:N_NODES, :1].set(ad2)

    acc2 = _edge_pass(1, 64, table2, src_p, dst_p, ad2p, zpad,
                      split_edges=False)

    out = pl.pallas_call(
        _final_body,
        out_shape=jax.ShapeDtypeStruct((N_NODES, FEAT), f32),
    )(acc2[0, :N_NODES], acc2[1, :N_NODES], p0, p1, b2.reshape(1, -1))
    return out


# speedup vs baseline: 1.2464x; 1.1286x over previous
import functools

import jax
import jax.numpy as jnp
from jax import lax
from jax.experimental import pallas as pl
from jax.experimental.pallas import tpu as pltpu
from jax.experimental.pallas import tpu_sc as plsc

N_NODES = 10000
FEAT = 128
HEADS = 8
CH = 8
E_EDGES = 320000

NC = 2
NS = 16

N_PAD = 10112
E_PAD = 327680
CHUNK = 64
ROWS_PT = N_PAD // NS
D = 128

_HI = lax.Precision.HIGHEST

_GDN = lax.GatherDimensionNumbers(
    offset_dims=(), collapsed_slice_dims=(0,), start_index_map=(0,))


def _vperm(v, idx):
    return lax.gather(v, idx[:, None], dimension_numbers=_GDN, slice_sizes=(1,),
                      mode=lax.GatherScatterMode.PROMISE_IN_BOUNDS)


def _dense1_body(x_ref, w1_ref, g1_ref, md_ref, table_ref, ad_ref):
    h = jnp.dot(x_ref[...], w1_ref[...], precision=_HI,
                preferred_element_type=jnp.float32)
    table_ref[...] = jnp.dot(h, g1_ref[...], precision=_HI,
                             preferred_element_type=jnp.float32)
    ad_ref[...] = jnp.dot(h, md_ref[...], precision=_HI,
                          preferred_element_type=jnp.float32)


def _dense2_body(acca_ref, accb_ref, rsel_ref, b1_ref, w2_ref, g2a_ref, g2b_ref,
                 a2d_ref, ta_ref, tb_ref, ad_ref):
    a = acca_ref[...] + accb_ref[...]
    srep = jnp.dot(a, rsel_ref[...], precision=_HI,
                   preferred_element_type=jnp.float32)
    m = a[:, :HEADS * CH] / (srep + 1e-16) + b1_ref[...]
    m = jnp.where(m > 0, m, jnp.exp(m) - 1.0)
    h2 = jnp.dot(m, w2_ref[...], precision=_HI,
                 preferred_element_type=jnp.float32)
    ta_ref[...] = jnp.dot(h2, g2a_ref[...], precision=_HI,
                          preferred_element_type=jnp.float32)
    tb_ref[...] = jnp.dot(h2, g2b_ref[...], precision=_HI,
                          preferred_element_type=jnp.float32)
    ad_ref[...] = jnp.dot(h2, a2d_ref[...], precision=_HI,
                          preferred_element_type=jnp.float32)


def _final_body(acca_ref, accb_ref, p0_ref, p1_ref, b2_ref, out_ref):
    a0 = acca_ref[...]
    a1 = accb_ref[...]
    o0 = a0[:, :64] / (a0[:, 64:65] + 1e-16)
    o1 = a1[:, :64] / (a1[:, 64:65] + 1e-16)
    out_ref[...] = (jnp.dot(o0, p0_ref[...], precision=_HI,
                            preferred_element_type=jnp.float32)
                    + jnp.dot(o1, p1_ref[...], precision=_HI,
                              preferred_element_type=jnp.float32)
                    + b2_ref[...])


def _edge_pass(H, C, table, src_p, dst_p, a_d, zeros, split_edges):
    mesh = plsc.VectorSubcoreMesh(core_axis_name="c", subcore_axis_name="s")
    nwork = NC * NS if split_edges else NS
    ept = E_PAD // nwork
    nchunks = ept // CHUNK
    assert nchunks % 2 == 0

    @functools.partial(
        pl.kernel, mesh=mesh,
        out_type=jax.ShapeDtypeStruct((NC, N_PAD, D), jnp.float32),
        scratch_types=[
            pltpu.VMEM((CHUNK,), jnp.int32), pltpu.VMEM((CHUNK,), jnp.int32),
            pltpu.VMEM((CHUNK,), jnp.int32), pltpu.VMEM((CHUNK,), jnp.int32),
            pltpu.VMEM((CHUNK, D), jnp.float32), pltpu.VMEM((CHUNK, D), jnp.float32),
            pltpu.VMEM((CHUNK, D), jnp.float32), pltpu.VMEM((CHUNK, D), jnp.float32),
            pltpu.VMEM_SHARED((N_PAD, D), jnp.float32),
            pltpu.SemaphoreType.DMA, pltpu.SemaphoreType.DMA,
            pltpu.SemaphoreType.DMA, pltpu.SemaphoreType.DMA,
        ],
    )
    def k(table_h, src_h, dst_h, ad_h, z_h, out_h,
          s0, s1, d0, d1, r0, r1, a0, a1, acc, g0, g1, e0, e1):
        srcb, dstb, rowsb, adrb = [s0, s1], [d0, d1], [r0, r1], [a0, a1]
        gsem, ssem = [g0, g1], [e0, e1]
        cid = lax.axis_index("c")
        sid = lax.axis_index("s")
        wid = cid * NS + sid if split_edges else sid
        off = cid * N_NODES

        def stage(c, b):
            base = pl.multiple_of(wid * ept + c * CHUNK, CHUNK)
            pltpu.sync_copy(src_h.at[pl.ds(base, CHUNK)], srcb[b])
            pltpu.sync_copy(dst_h.at[pl.ds(base, CHUNK)], dstb[b])
            if not split_edges:
                for g in range(CHUNK // 16):
                    srcb[b][pl.ds(g * 16, 16)] = srcb[b][pl.ds(g * 16, 16)] + off
            pltpu.async_copy(table_h.at[srcb[b]], rowsb[b], gsem[b])
            pltpu.async_copy(ad_h.at[dstb[b]], adrb[b], gsem[b])

        def wait_gathers(b):
            pltpu.make_async_copy(table_h.at[srcb[b]], rowsb[b], gsem[b]).wait()
            pltpu.make_async_copy(ad_h.at[dstb[b]], adrb[b], gsem[b]).wait()

        def wait_scatter(b):
            pltpu.make_async_copy(rowsb[b], acc.at[dstb[b]], ssem[b]).wait()

        row0 = pl.multiple_of(sid * ROWS_PT, 8)
        pltpu.sync_copy(z_h.at[pl.ds(row0, ROWS_PT)],
                        acc.at[pl.ds(row0, ROWS_PT)])
        plsc.subcore_barrier()
        log2c = C.bit_length() - 1
        stage(0, 0)

        def body(cc, carry):
            iota = lax.iota(jnp.int32, 16)
            pmask = jnp.where(iota < H, 1.0, 0.0)
            pidx = [(iota + 16 * j) >> log2c for j in range(H * C // 16)]
            for b in (0, 1):
                c = 2 * cc + b
                nb = 1 - b
                if b == 0:
                    @pl.when(cc > 0)
                    def _():
                        wait_scatter(nb)
                else:
                    wait_scatter(nb)
                stage(c + 1, nb)
                wait_gathers(b)
                rows_v, adr_v = rowsb[b], adrb[b]
                for i in range(CHUNK):
                    ad = adr_v[i, pl.ds(0, 16)]
                    as_ = rows_v[i, pl.ds(H * C, 16)]
                    e = as_ + ad
                    e = jnp.where(e > 0, e, 0.2 * e)
                    p = jnp.exp(e) * pmask
                    rows_v[i, pl.ds(H * C, 16)] = p
                    for j in range(H * C // 16):
                        pv = _vperm(p, pidx[j])
                        v = rows_v[i, pl.ds(16 * j, 16)]
                        rows_v[i, pl.ds(16 * j, 16)] = v * pv
                pltpu.async_copy(rows_v, acc.at[dstb[b]], ssem[b], add=True)
            return carry

        lax.fori_loop(0, nchunks // 2, body, 0)
        wait_scatter(1)
        wait_gathers(0)
        plsc.subcore_barrier()
        pltpu.sync_copy(acc.at[pl.ds(row0, ROWS_PT)],
                        out_h.at[cid, pl.ds(row0, ROWS_PT)])

    return k(table, src_p, dst_p, a_d, zeros)


def kernel(x, edge_index, W1, att_src1, att_dst1, b1, W2, att_src2, att_dst2, b2):
    f32 = jnp.float32
    src = edge_index[0]
    dst = edge_index[1]
    pad_e = E_PAD + CHUNK - E_EDGES
    src_p = jnp.concatenate([src, jnp.zeros((pad_e,), jnp.int32)])
    dst_p = jnp.concatenate(
        [dst, N_NODES + (jnp.arange(pad_e, dtype=jnp.int32) % 100)])

    eyeH = jnp.eye(HEADS, dtype=f32)
    eye64 = jnp.eye(64, dtype=f32)
    m1s = (att_src1[:, :, None] * eyeH[:, None, :]).reshape(HEADS * CH, HEADS)
    m1d = (att_dst1[:, :, None] * eyeH[:, None, :]).reshape(HEADS * CH, HEADS)
    g1 = jnp.concatenate([eye64, m1s, jnp.zeros((64, 56), f32)], axis=1)
    rsel = jnp.concatenate(
        [jnp.zeros((64, 64), f32),
         jnp.repeat(eyeH, CH, axis=1),
         jnp.zeros((56, 64), f32)], axis=0)
    top = jnp.concatenate([eye64, jnp.zeros((64, 64), f32)], axis=0)
    bot = jnp.concatenate([jnp.zeros((64, 64), f32), eye64], axis=0)
    g2a = jnp.concatenate([top, att_src2.T, jnp.zeros((FEAT, 63), f32)], axis=1)
    g2b = jnp.concatenate([bot, att_src2.T, jnp.zeros((FEAT, 63), f32)], axis=1)
    a2d = att_dst2.T
    p0 = top.T
    p1 = bot.T

    table1, ad1 = pl.pallas_call(
        _dense1_body,
        out_shape=(jax.ShapeDtypeStruct((N_NODES, D), f32),
                   jax.ShapeDtypeStruct((N_NODES, HEADS), f32)),
    )(x, W1, g1, m1d)
    ad1p = jnp.zeros((N_PAD, D), f32).at[:N_NODES, :HEADS].set(ad1)
    zpad = jnp.zeros((N_PAD, D), f32)

    acc1 = _edge_pass(HEADS, CH, table1, src_p, dst_p, ad1p, zpad,
                      split_edges=True)

    t2a, t2b, ad2 = pl.pallas_call(
        _dense2_body,
        out_shape=(jax.ShapeDtypeStruct((N_NODES, D), f32),
                   jax.ShapeDtypeStruct((N_NODES, D), f32),
                   jax.ShapeDtypeStruct((N_NODES, 1), f32)),
    )(acc1[0, :N_NODES], acc1[1, :N_NODES], rsel, b1.reshape(1, -1), W2,
      g2a, g2b, a2d)
    table2 = jnp.concatenate([t2a, t2b], axis=0)
    ad2p = jnp.zeros((N_PAD, D), f32).at[:N_NODES, :1].set(ad2)

    acc2 = _edge_pass(1, 64, table2, src_p, dst_p, ad2p, zpad,
                      split_edges=False)

    out = pl.pallas_call(
        _final_body,
        out_shape=jax.ShapeDtypeStruct((N_NODES, FEAT), f32),
    )(acc2[0, :N_NODES], acc2[1, :N_NODES], p0, p1, b2.reshape(1, -1))
    return out


# speedup vs baseline: 1.3276x; 1.0652x over previous
import functools

import jax
import jax.numpy as jnp
from jax import lax
from jax.experimental import pallas as pl
from jax.experimental.pallas import tpu as pltpu
from jax.experimental.pallas import tpu_sc as plsc

N_NODES = 10000
FEAT = 128
HEADS = 8
CH = 8
E_EDGES = 320000

NC = 2
NS = 16

N_PAD = 10112
E_PAD = 327680
CHUNK = 64
ROWS_PT = N_PAD // NS
D = 128

_HI = lax.Precision.HIGHEST

_GDN = lax.GatherDimensionNumbers(
    offset_dims=(), collapsed_slice_dims=(0,), start_index_map=(0,))


def _vperm(v, idx):
    return lax.gather(v, idx[:, None], dimension_numbers=_GDN, slice_sizes=(1,),
                      mode=lax.GatherScatterMode.PROMISE_IN_BOUNDS)


def _dense1_body(x_ref, w1_ref, g1_ref, md_ref, table_ref, ad_ref):
    h = jnp.dot(x_ref[...], w1_ref[...], precision=_HI,
                preferred_element_type=jnp.float32)
    table_ref[...] = jnp.dot(h, g1_ref[...], precision=_HI,
                             preferred_element_type=jnp.float32)
    ad_ref[...] = jnp.dot(h, md_ref[...], precision=_HI,
                          preferred_element_type=jnp.float32)


def _dense2_body(acca_ref, accb_ref, rsel_ref, b1_ref, w2_ref, g2a_ref, g2b_ref,
                 a2d_ref, ta_ref, tb_ref, ad_ref):
    a = acca_ref[...] + accb_ref[...]
    srep = jnp.dot(a, rsel_ref[...], precision=_HI,
                   preferred_element_type=jnp.float32)
    m = a[:, :HEADS * CH] / (srep + 1e-16) + b1_ref[...]
    m = jnp.where(m > 0, m, jnp.exp(m) - 1.0)
    h2 = jnp.dot(m, w2_ref[...], precision=_HI,
                 preferred_element_type=jnp.float32)
    ta_ref[...] = jnp.dot(h2, g2a_ref[...], precision=_HI,
                          preferred_element_type=jnp.float32)
    tb_ref[...] = jnp.dot(h2, g2b_ref[...], precision=_HI,
                          preferred_element_type=jnp.float32)
    ad_ref[...] = jnp.dot(h2, a2d_ref[...], precision=_HI,
                          preferred_element_type=jnp.float32)


def _final_body(acca_ref, accb_ref, p0_ref, p1_ref, b2_ref, out_ref):
    a0 = acca_ref[...]
    a1 = accb_ref[...]
    o0 = a0[:, :64] / (a0[:, 64:65] + 1e-16)
    o1 = a1[:, :64] / (a1[:, 64:65] + 1e-16)
    out_ref[...] = (jnp.dot(o0, p0_ref[...], precision=_HI,
                            preferred_element_type=jnp.float32)
                    + jnp.dot(o1, p1_ref[...], precision=_HI,
                              preferred_element_type=jnp.float32)
                    + b2_ref[...])


def _edge_pass(H, C, table, cidx, zeros, split_edges):
    mesh = plsc.VectorSubcoreMesh(core_axis_name="c", subcore_axis_name="s")
    nwork = NC * NS if split_edges else NS
    ept = E_PAD // nwork
    nchunks = ept // CHUNK
    assert nchunks % 2 == 0

    adbase = (NC if not split_edges else 1) * N_NODES

    @functools.partial(
        pl.kernel, mesh=mesh,
        out_type=jax.ShapeDtypeStruct((NC, N_PAD, D), jnp.float32),
        scratch_types=[
            pltpu.VMEM((2 * CHUNK,), jnp.int32), pltpu.VMEM((2 * CHUNK,), jnp.int32),
            pltpu.VMEM((CHUNK,), jnp.int32), pltpu.VMEM((CHUNK,), jnp.int32),
            pltpu.VMEM((2 * CHUNK, D), jnp.float32),
            pltpu.VMEM((2 * CHUNK, D), jnp.float32),
            pltpu.VMEM_SHARED((N_PAD, D), jnp.float32),
            pltpu.SemaphoreType.DMA, pltpu.SemaphoreType.DMA,
            pltpu.SemaphoreType.DMA, pltpu.SemaphoreType.DMA,
        ],
    )
    def k(table_h, cidx_h, z_h, out_h,
          s0, s1, d0, d1, r0, r1, acc, g0, g1, e0, e1):
        idxb, dstb, rowsb = [s0, s1], [d0, d1], [r0, r1]
        gsem, ssem = [g0, g1], [e0, e1]
        cid = lax.axis_index("c")
        sid = lax.axis_index("s")
        wid = sid * NC + cid if split_edges else sid
        off = cid * N_NODES

        def stage(c, b):
            base = pl.multiple_of((wid * ept + c * CHUNK) * 2, 2 * CHUNK)
            pltpu.sync_copy(cidx_h.at[pl.ds(base, 2 * CHUNK)], idxb[b])
            if not split_edges:
                for g in range(CHUNK // 16):
                    idxb[b][pl.ds(g * 16, 16)] = idxb[b][pl.ds(g * 16, 16)] + off
            pltpu.async_copy(table_h.at[idxb[b]], rowsb[b], gsem[b])

        def wait_gathers(b):
            pltpu.make_async_copy(table_h.at[idxb[b]], rowsb[b], gsem[b]).wait()

        def wait_scatter(b):
            pltpu.make_async_copy(rowsb[b].at[pl.ds(0, CHUNK)], acc.at[dstb[b]],
                                  ssem[b]).wait()

        row0 = pl.multiple_of(sid * ROWS_PT, 8)
        pltpu.sync_copy(z_h.at[pl.ds(row0, ROWS_PT)],
                        acc.at[pl.ds(row0, ROWS_PT)])
        plsc.subcore_barrier()
        log2c = C.bit_length() - 1
        stage(0, 0)

        def body(cc, carry):
            iota = lax.iota(jnp.int32, 16)
            pmask = jnp.where(iota < H, 1.0, 0.0)
            pidx = [(iota + 16 * j) >> log2c for j in range(H * C // 16)]
            for b in (0, 1):
                c = 2 * cc + b
                nb = 1 - b
                if b == 0:
                    @pl.when(cc > 0)
                    def _():
                        wait_scatter(nb)
                else:
                    wait_scatter(nb)
                stage(c + 1, nb)
                for g in range(CHUNK // 16):
                    dstb[b][pl.ds(g * 16, 16)] = (
                        idxb[b][pl.ds(CHUNK + g * 16, 16)] - adbase)
                wait_gathers(b)
                rows_v = rowsb[b]
                for i in range(CHUNK):
                    ad = rows_v[CHUNK + i, pl.ds(0, 16)]
                    as_ = rows_v[i, pl.ds(H * C, 16)]
                    e = as_ + ad
                    e = jnp.where(e > 0, e, 0.2 * e)
                    p = jnp.exp(e) * pmask
                    rows_v[i, pl.ds(H * C, 16)] = p
                    for j in range(H * C // 16):
                        pv = _vperm(p, pidx[j])
                        v = rows_v[i, pl.ds(16 * j, 16)]
                        rows_v[i, pl.ds(16 * j, 16)] = v * pv
                pltpu.async_copy(rows_v.at[pl.ds(0, CHUNK)], acc.at[dstb[b]],
                                 ssem[b], add=True)
            return carry

        lax.fori_loop(0, nchunks // 2, body, 0)
        wait_scatter(1)
        wait_gathers(0)
        plsc.subcore_barrier()
        pltpu.sync_copy(acc.at[pl.ds(row0, ROWS_PT)],
                        out_h.at[cid, pl.ds(row0, ROWS_PT)])

    return k(table, cidx, zeros)


def kernel(x, edge_index, W1, att_src1, att_dst1, b1, W2, att_src2, att_dst2, b2):
    f32 = jnp.float32
    src = edge_index[0]
    dst = edge_index[1]
    pad_e = E_PAD + CHUNK - E_EDGES
    src_p = jnp.concatenate([src, jnp.zeros((pad_e,), jnp.int32)])
    dst_p = jnp.concatenate(
        [dst, N_NODES + (jnp.arange(pad_e, dtype=jnp.int32) % 100)])

    def combined_idx(adbase):
        sb = src_p.reshape(-1, CHUNK)
        db = dst_p.reshape(-1, CHUNK) + adbase
        return jnp.concatenate([sb, db], axis=1).reshape(-1)

    cidx1 = combined_idx(N_NODES)
    cidx2 = combined_idx(2 * N_NODES)

    eyeH = jnp.eye(HEADS, dtype=f32)
    eye64 = jnp.eye(64, dtype=f32)
    m1s = (att_src1[:, :, None] * eyeH[:, None, :]).reshape(HEADS * CH, HEADS)
    m1d = (att_dst1[:, :, None] * eyeH[:, None, :]).reshape(HEADS * CH, HEADS)
    g1 = jnp.concatenate([eye64, m1s, jnp.zeros((64, 56), f32)], axis=1)
    rsel = jnp.concatenate(
        [jnp.zeros((64, 64), f32),
         jnp.repeat(eyeH, CH, axis=1),
         jnp.zeros((56, 64), f32)], axis=0)
    top = jnp.concatenate([eye64, jnp.zeros((64, 64), f32)], axis=0)
    bot = jnp.concatenate([jnp.zeros((64, 64), f32), eye64], axis=0)
    g2a = jnp.concatenate([top, att_src2.T, jnp.zeros((FEAT, 63), f32)], axis=1)
    g2b = jnp.concatenate([bot, att_src2.T, jnp.zeros((FEAT, 63), f32)], axis=1)
    a2d = att_dst2.T
    p0 = top.T
    p1 = bot.T

    table1, ad1 = pl.pallas_call(
        _dense1_body,
        out_shape=(jax.ShapeDtypeStruct((N_NODES, D), f32),
                   jax.ShapeDtypeStruct((N_NODES, HEADS), f32)),
    )(x, W1, g1, m1d)
    ad1p = jnp.zeros((N_PAD, D), f32).at[:N_NODES, :HEADS].set(ad1)
    zpad = jnp.zeros((N_PAD, D), f32)

    acc1 = _edge_pass(HEADS, CH, jnp.concatenate([table1, ad1p], axis=0),
                      cidx1, zpad, split_edges=True)

    t2a, t2b, ad2 = pl.pallas_call(
        _dense2_body,
        out_shape=(jax.ShapeDtypeStruct((N_NODES, D), f32),
                   jax.ShapeDtypeStruct((N_NODES, D), f32),
                   jax.ShapeDtypeStruct((N_NODES, 1), f32)),
    )(acc1[0, :N_NODES], acc1[1, :N_NODES], rsel, b1.reshape(1, -1), W2,
      g2a, g2b, a2d)
    ad2p = jnp.zeros((N_PAD, D), f32).at[:N_NODES, :1].set(ad2)
    table2 = jnp.concatenate([t2a, t2b, ad2p], axis=0)

    acc2 = _edge_pass(1, 64, table2, cidx2, zpad, split_edges=False)

    out = pl.pallas_call(
        _final_body,
        out_shape=jax.ShapeDtypeStruct((N_NODES, FEAT), f32),
    )(acc2[0, :N_NODES], acc2[1, :N_NODES], p0, p1, b2.reshape(1, -1))
    return out
